# baseline (speedup 1.0000x reference)
import functools

import jax
import jax.numpy as jnp
from jax import lax
from jax.experimental import pallas as pl
from jax.experimental.pallas import tpu as pltpu
from jax.experimental.pallas import tpu_sc as plsc

S = 64.0
MARGIN = 0.5
_COS_M = 0.8775825618903728
_SIN_M = 0.479425538604203

_LANES = 16


def _sc_sqrt(y):
  i = plsc.bitcast(y, jnp.int32)
  r = plsc.bitcast(0x5F3759DF - (i >> 1), jnp.float32)
  for _ in range(3):
    r = r * (1.5 - 0.5 * y * r * r)
  return y * r


def _sc_stream(logits, labels, B, C):
  info = plsc.get_sparse_core_info()
  nw = info.num_cores * info.num_subcores
  bands_per_w = (B // 8) // nw
  rows_per_w = B // nw

  wmain = 6144
  nmain = 16
  wrem = C - C % 128 - nmain * wmain
  wtail = C % 128
  mesh = plsc.VectorSubcoreMesh(core_axis_name="c", subcore_axis_name="s")

  @functools.partial(
      pl.kernel,
      out_type=jax.ShapeDtypeStruct((B, C), jnp.float32),
      mesh=mesh,
      compiler_params=pltpu.CompilerParams(needs_layout_passes=False),
      scratch_types=[
          pltpu.VMEM((8, wmain), jnp.float32),
          pltpu.VMEM((8, wrem), jnp.float32),
          pltpu.VMEM((8, wtail), jnp.float32),
          pltpu.VMEM((rows_per_w,), jnp.int32),
      ],
  )
  def k(x_hbm, lab_hbm, o_hbm, buf, buf_r, buf_t, lab_v):
    wid = lax.axis_index("s") * info.num_cores + lax.axis_index("c")
    base_row = wid * rows_per_w

    pltpu.sync_copy(lab_hbm.at[pl.ds(base_row, rows_per_w)], lab_v)
    lane = lax.iota(jnp.int32, _LANES)
    rowlane = lane & 7
    is_row = lane < 8

    def fixup(bi, bref, c0, w):
      lab16 = plsc.load_gather(lab_v, [bi * 8 + rowlane])
      lab_loc = lab16 - c0
      inrange = (lab_loc >= 0) & (lab_loc < w) & is_row
      safe_loc = jnp.where(inrange, lab_loc, 0)
      t = plsc.load_gather(bref, [rowlane, safe_loc], mask=inrange)
      fv = (t * _COS_M - _sc_sqrt(jnp.maximum(1.0 - t * t, 0.0)) * _SIN_M)
      plsc.store_scatter(bref, [rowlane, safe_loc], fv, mask=inrange)

    def scale(bref, w):
      def row_body(r, c):
        def col_body(v, c2):
          sl = pl.ds(v * (4 * _LANES), _LANES)
          for u in range(4):
            slu = pl.ds(v * (4 * _LANES) + u * _LANES, _LANES)
            bref[r, slu] = bref[r, slu] * S
          return c2
        lax.fori_loop(0, w // (4 * _LANES), col_body, 0)
        return c
      lax.fori_loop(0, 8, row_body, 0)

    def do_band(bi, carry):
      r0 = base_row + bi * 8

      def do_chunk(mi, c):
        c0 = mi * wmain
        pltpu.sync_copy(x_hbm.at[pl.ds(r0, 8), pl.ds(c0, wmain)], buf)
        fixup(bi, buf, c0, wmain)
        scale(buf, wmain)
        pltpu.sync_copy(buf, o_hbm.at[pl.ds(r0, 8), pl.ds(c0, wmain)])
        return c

      lax.fori_loop(0, nmain, do_chunk, 0)

      c0 = nmain * wmain
      pltpu.sync_copy(x_hbm.at[pl.ds(r0, 8), pl.ds(c0, wrem)], buf_r)
      fixup(bi, buf_r, c0, wrem)
      scale(buf_r, wrem)
      pltpu.sync_copy(buf_r, o_hbm.at[pl.ds(r0, 8), pl.ds(c0, wrem)])

      c0 = C - wtail
      pltpu.sync_copy(x_hbm.at[pl.ds(r0, 8), pl.ds(c0, wtail)], buf_t)
      fixup(bi, buf_t, c0, wtail)
      for r in range(8):
        for u in range(wtail // _LANES):
          sl = pl.ds(u * _LANES, _LANES)
          buf_t[r, sl] = buf_t[r, sl] * S
      pltpu.sync_copy(buf_t, o_hbm.at[pl.ds(r0, 8), pl.ds(c0, wtail)])
      return carry

    lax.fori_loop(0, bands_per_w, do_band, 0)

  return k(logits, labels)


def kernel(logits, labels, embeddings):
  B, C = logits.shape
  out = _sc_stream(logits, labels, B, C)
  return (out, None)


# speedup vs baseline: 1.8338x; 1.8338x over previous
import functools

import jax
import jax.numpy as jnp
from jax import lax
from jax.experimental import pallas as pl
from jax.experimental.pallas import tpu as pltpu
from jax.experimental.pallas import tpu_sc as plsc

S = 64.0
MARGIN = 0.5
_COS_M = 0.8775825618903728
_SIN_M = 0.479425538604203

_LANES = 16


def _sc_sqrt(y):
  i = plsc.bitcast(y, jnp.int32)
  r = plsc.bitcast(0x5F3759DF - (i >> 1), jnp.float32)
  for _ in range(3):
    r = r * (1.5 - 0.5 * y * r * r)
  return y * r


def _sc_stream(logits, labels, B, C):
  info = plsc.get_sparse_core_info()
  nw = info.num_cores * info.num_subcores
  bands_per_w = (B // 8) // nw
  rows_per_w = B // nw

  wmain = 6144
  nmain = 16
  wrem = C - C % 128 - nmain * wmain
  wtail = C % 128
  mesh = plsc.VectorSubcoreMesh(core_axis_name="c", subcore_axis_name="s")

  @functools.partial(
      pl.kernel,
      out_type=jax.ShapeDtypeStruct((B, C), jnp.float32),
      mesh=mesh,
      compiler_params=pltpu.CompilerParams(needs_layout_passes=False),
      scratch_types=[
          pltpu.VMEM((8, wmain), jnp.float32),
          pltpu.VMEM((8, wrem), jnp.float32),
          pltpu.VMEM((8, wtail), jnp.float32),
          pltpu.VMEM((rows_per_w,), jnp.int32),
      ],
  )
  def k(x_hbm, lab_hbm, o_hbm, buf, buf_r, buf_t, lab_v):
    wid = lax.axis_index("s") * info.num_cores + lax.axis_index("c")
    base_row = wid * rows_per_w

    pltpu.sync_copy(lab_hbm.at[pl.ds(base_row, rows_per_w)], lab_v)
    lane = lax.iota(jnp.int32, _LANES)
    rowlane = lane & 7
    is_row = lane < 8

    def fixup(bi, bref, c0, w):
      lab16 = plsc.load_gather(lab_v, [bi * 8 + rowlane])
      lab_loc = lab16 - c0
      inrange = (lab_loc >= 0) & (lab_loc < w) & is_row
      safe_loc = jnp.where(inrange, lab_loc, 0)
      t = plsc.load_gather(bref, [rowlane, safe_loc], mask=inrange)
      fv = (t * _COS_M - _sc_sqrt(jnp.maximum(1.0 - t * t, 0.0)) * _SIN_M)
      plsc.store_scatter(bref, [rowlane, safe_loc], fv, mask=inrange)

    def scale(bref, w):
      for r in range(8):
        @plsc.parallel_loop(0, w // _LANES, step=1, unroll=8)
        def _scale_row(v):
          sl = pl.ds(v * _LANES, _LANES)
          bref[r, sl] = bref[r, sl] * S

    def do_band(bi, carry):
      r0 = base_row + bi * 8

      def do_chunk(mi, c):
        c0 = mi * wmain
        pltpu.sync_copy(x_hbm.at[pl.ds(r0, 8), pl.ds(c0, wmain)], buf)
        fixup(bi, buf, c0, wmain)
        scale(buf, wmain)
        pltpu.sync_copy(buf, o_hbm.at[pl.ds(r0, 8), pl.ds(c0, wmain)])
        return c

      lax.fori_loop(0, nmain, do_chunk, 0)

      c0 = nmain * wmain
      pltpu.sync_copy(x_hbm.at[pl.ds(r0, 8), pl.ds(c0, wrem)], buf_r)
      fixup(bi, buf_r, c0, wrem)
      scale(buf_r, wrem)
      pltpu.sync_copy(buf_r, o_hbm.at[pl.ds(r0, 8), pl.ds(c0, wrem)])

      c0 = C - wtail
      pltpu.sync_copy(x_hbm.at[pl.ds(r0, 8), pl.ds(c0, wtail)], buf_t)
      fixup(bi, buf_t, c0, wtail)
      for r in range(8):
        for u in range(wtail // _LANES):
          sl = pl.ds(u * _LANES, _LANES)
          buf_t[r, sl] = buf_t[r, sl] * S
      pltpu.sync_copy(buf_t, o_hbm.at[pl.ds(r0, 8), pl.ds(c0, wtail)])
      return carry

    lax.fori_loop(0, bands_per_w, do_band, 0)

  return k(logits, labels)


def kernel(logits, labels, embeddings):
  B, C = logits.shape
  out = _sc_stream(logits, labels, B, C)
  return (out, None)


# speedup vs baseline: 2.0878x; 1.1385x over previous
import functools

import jax
import jax.numpy as jnp
from jax import lax
from jax.experimental import pallas as pl
from jax.experimental.pallas import tpu as pltpu
from jax.experimental.pallas import tpu_sc as plsc

S = 64.0
MARGIN = 0.5
_COS_M = 0.8775825618903728
_SIN_M = 0.479425538604203

_LANES = 16
_NBUF = 4
_WMAIN = 3072
_NMAIN = 32


def _sc_sqrt(y):
  i = plsc.bitcast(y, jnp.int32)
  r = plsc.bitcast(0x5F3759DF - (i >> 1), jnp.float32)
  for _ in range(3):
    r = r * (1.5 - 0.5 * y * r * r)
  return y * r


def _sc_stream(logits, labels, B, C):
  info = plsc.get_sparse_core_info()
  nw = info.num_cores * info.num_subcores
  rows_per_w = B // nw
  bands_per_w = rows_per_w // 8
  total = bands_per_w * _NMAIN

  wrem = C - C % 128 - _NMAIN * _WMAIN
  wtail = C % 128
  c_rem = _NMAIN * _WMAIN
  c_tail = C - wtail
  mesh = plsc.VectorSubcoreMesh(core_axis_name="c", subcore_axis_name="s")

  @functools.partial(
      pl.kernel,
      out_type=jax.ShapeDtypeStruct((B, C), jnp.float32),
      mesh=mesh,
      compiler_params=pltpu.CompilerParams(needs_layout_passes=False),
      scratch_types=[
          [pltpu.VMEM((8, _WMAIN), jnp.float32) for _ in range(_NBUF)],
          pltpu.VMEM((8, wrem), jnp.float32),
          pltpu.VMEM((8, wtail), jnp.float32),
          pltpu.VMEM((rows_per_w,), jnp.int32),
          [pltpu.SemaphoreType.DMA for _ in range(_NBUF)],
          [pltpu.SemaphoreType.DMA for _ in range(_NBUF)],
      ],
  )
  def k(x_hbm, lab_hbm, o_hbm, bufs, buf_r, buf_t, lab_v, in_sems, out_sems):
    wid = lax.axis_index("s") * info.num_cores + lax.axis_index("c")
    base_row = wid * rows_per_w

    pltpu.sync_copy(lab_hbm.at[pl.ds(base_row, rows_per_w)], lab_v)
    lane = lax.iota(jnp.int32, _LANES)
    rowlane = lane & 7
    is_row = lane < 8

    def slab(g):
      r0 = base_row + (g >> 5) * 8
      c0 = (g & (_NMAIN - 1)) * _WMAIN
      return r0, c0

    def in_copy(g, b):
      r0, c0 = slab(g)
      return pltpu.make_async_copy(
          x_hbm.at[pl.ds(r0, 8), pl.ds(c0, _WMAIN)], bufs[b], in_sems[b])

    def out_copy(g, b):
      r0, c0 = slab(g)
      return pltpu.make_async_copy(
          bufs[b], o_hbm.at[pl.ds(r0, 8), pl.ds(c0, _WMAIN)], out_sems[b])

    def fixup(bi, bref, c0, w):
      lab16 = plsc.load_gather(lab_v, [bi * 8 + rowlane])
      lab_loc = lab16 - c0
      inrange = (lab_loc >= 0) & (lab_loc < w) & is_row
      safe_loc = jnp.where(inrange, lab_loc, 0)
      t = plsc.load_gather(bref, [rowlane, safe_loc], mask=inrange)
      fv = (t * _COS_M - _sc_sqrt(jnp.maximum(1.0 - t * t, 0.0)) * _SIN_M)
      plsc.store_scatter(bref, [rowlane, safe_loc], fv, mask=inrange)

    def scale(bref, w):
      for r in range(8):
        @plsc.parallel_loop(0, w // _LANES, step=1, unroll=8)
        def _scale_row(v):
          sl = pl.ds(v * _LANES, _LANES)
          bref[r, sl] = bref[r, sl] * S

    for b in range(_NBUF - 1):
      in_copy(b, b).start()

    def do_round(gg, carry):
      for b in range(_NBUF):
        g = gg * _NBUF + b
        in_copy(g, b).wait()
        fixup(g >> 5, bufs[b], (g & (_NMAIN - 1)) * _WMAIN, _WMAIN)
        scale(bufs[b], _WMAIN)
        out_copy(g, b).start()
        b3 = (b + _NBUF - 1) % _NBUF

        @pl.when(g + _NBUF - 1 < total)
        def _prefetch():
          @pl.when(g >= 1)
          def _drain():
            out_copy(g - 1, b3).wait()
          in_copy(g + _NBUF - 1, b3).start()
      return carry

    lax.fori_loop(0, total // _NBUF, do_round, 0)
    for b in range(_NBUF):
      out_copy(total - _NBUF + b, b).wait()

    def do_edges(bi, carry):
      r0 = base_row + bi * 8
      pltpu.sync_copy(x_hbm.at[pl.ds(r0, 8), pl.ds(c_rem, wrem)], buf_r)
      fixup(bi, buf_r, c_rem, wrem)
      scale(buf_r, wrem)
      pltpu.sync_copy(buf_r, o_hbm.at[pl.ds(r0, 8), pl.ds(c_rem, wrem)])

      pltpu.sync_copy(x_hbm.at[pl.ds(r0, 8), pl.ds(c_tail, wtail)], buf_t)
      fixup(bi, buf_t, c_tail, wtail)
      for r in range(8):
        for u in range(wtail // _LANES):
          sl = pl.ds(u * _LANES, _LANES)
          buf_t[r, sl] = buf_t[r, sl] * S
      pltpu.sync_copy(buf_t, o_hbm.at[pl.ds(r0, 8), pl.ds(c_tail, wtail)])
      return carry

    lax.fori_loop(0, bands_per_w, do_edges, 0)

  return k(logits, labels)


def kernel(logits, labels, embeddings):
  B, C = logits.shape
  out = _sc_stream(logits, labels, B, C)
  return (out, None)
